# Initial kernel scaffold; baseline (speedup 1.0000x reference)
import jax
import jax.numpy as jnp
from jax.experimental import pallas as pl


def kernel(x, w, gamma, beta):
    raise NotImplementedError("write your pallas kernel here")


# speedup vs baseline: 1.0600x; 1.0600x over previous
import jax
import jax.numpy as jnp
import numpy as np
from jax.experimental import pallas as pl
from jax.experimental.pallas import tpu as pltpu


def _weff_index(out_g, cin_g):
    rows, cols = 4 * out_g, cin_g * 16
    idx = np.zeros((rows, cols), np.int32)
    valid = np.zeros((rows, cols), bool)
    for dh in range(2):
        for dw in range(2):
            off = dh * 2 + dw
            for o in range(out_g):
                r = off * out_g + o
                for cl in range(cin_g):
                    for kh in range(3):
                        for kw in range(3):
                            st = (dh + kh) * 4 + (dw + kw)
                            c = cl * 16 + st
                            idx[r, c] = o * cin_g * 9 + cl * 9 + kh * 3 + kw
                            valid[r, c] = True
    return idx, valid


def _make_body(out_g, inv_count, eps):
    def _body(ph_ref, weff_ref, gamma_ref, beta_ref, o_ref):
        ph = ph_ref[0]
        wf = weff_ref[0]
        acc = jnp.dot(wf, ph, preferred_element_type=jnp.float32)

        rs = jnp.sum(acc, axis=1, keepdims=True)
        sch = (rs[0:out_g] + rs[out_g:2 * out_g]
               + rs[2 * out_g:3 * out_g] + rs[3 * out_g:4 * out_g])
        mean = sch * inv_count
        mean4 = jnp.concatenate([mean, mean, mean, mean], axis=0)
        d = acc - mean4
        rq = jnp.sum(d * d, axis=1, keepdims=True)
        var = (rq[0:out_g] + rq[out_g:2 * out_g]
               + rq[2 * out_g:3 * out_g] + rq[3 * out_g:4 * out_g]) * inv_count
        scale = gamma_ref[0] * jax.lax.rsqrt(var + eps)
        shift = beta_ref[0] - mean * scale
        scale4 = jnp.concatenate([scale, scale, scale, scale], axis=0)
        shift4 = jnp.concatenate([shift, shift, shift, shift], axis=0)

        y = jnp.maximum(acc * scale4 + shift4, 0.0)
        o_ref[0] = jnp.maximum(
            jnp.maximum(y[0:out_g], y[out_g:2 * out_g]),
            jnp.maximum(y[2 * out_g:3 * out_g], y[3 * out_g:4 * out_g]))

    return _body


def _forward(x, w, gamma, beta, groups, eps=1e-5):
    N, Cin, H, W = x.shape
    Cout = w.shape[0]
    H2, W2 = H // 2, W // 2
    G, cin_g, out_g = groups, Cin // groups, Cout // groups
    S = N * H2 * W2
    count = N * H * W

    xpad = jnp.pad(x.astype(jnp.float32), ((0, 0), (0, 0), (1, 1), (1, 1)))
    phase_list = [xpad[:, :, s:s + 2 * H2:2, t:t + 2 * W2:2]
                  for s in range(4) for t in range(4)]
    phases = jnp.transpose(jnp.stack(phase_list, 0), (2, 0, 1, 3, 4))
    phases = phases.reshape(G, cin_g * 16, S)

    idx, valid = _weff_index(out_g, cin_g)
    w_grp = w.astype(jnp.float32).reshape(G, out_g * cin_g * 9)
    w_eff = jnp.where(jnp.asarray(valid), w_grp[:, jnp.asarray(idx)], 0.0)

    gamma_c = gamma.astype(jnp.float32).reshape(G, out_g, 1)
    beta_c = beta.astype(jnp.float32).reshape(G, out_g, 1)

    body = _make_body(out_g, 1.0 / float(count), eps)

    flops = 2 * S * 4 * Cout * cin_g * 16 + 8 * count * Cout
    bytes_accessed = phases.size * 4 + Cout * S * 4
    cost = pl.CostEstimate(flops=flops, transcendentals=Cout,
                           bytes_accessed=int(bytes_accessed))

    out_gm = pl.pallas_call(
        body,
        grid=(G,),
        in_specs=[
            pl.BlockSpec((1, cin_g * 16, S), lambda g: (g, 0, 0)),
            pl.BlockSpec((1, 4 * out_g, cin_g * 16), lambda g: (g, 0, 0)),
            pl.BlockSpec((1, out_g, 1), lambda g: (g, 0, 0)),
            pl.BlockSpec((1, out_g, 1), lambda g: (g, 0, 0)),
        ],
        out_specs=pl.BlockSpec((1, out_g, S), lambda g: (g, 0, 0)),
        out_shape=jax.ShapeDtypeStruct((G, out_g, S), jnp.float32),
        compiler_params=pltpu.CompilerParams(
            dimension_semantics=("parallel",),
            vmem_limit_bytes=60 * 1024 * 1024),
        cost_estimate=cost,
    )(phases, w_eff, gamma_c, beta_c)

    out = out_gm.reshape(Cout, N, H2, W2)
    return jnp.transpose(out, (1, 0, 2, 3))


def kernel(x, w, gamma, beta):
    return _forward(x, w, gamma, beta, groups=8)


# speedup vs baseline: 25.5238x; 24.0781x over previous
import jax
import jax.numpy as jnp
import numpy as np
from jax.experimental import pallas as pl
from jax.experimental.pallas import tpu as pltpu


_SPLIT = {0: (1, -1), 1: (0, 0), 2: (1, 0), 3: (0, 1)}


def _weff_index(out_g, cin_g):
    rows, cols = 4 * out_g, cin_g * 16
    idx = np.zeros((rows, cols), np.int32)
    valid = np.zeros((rows, cols), bool)
    for dh in range(2):
        for dw in range(2):
            off = dh * 2 + dw
            for o in range(out_g):
                r = off * out_g + o
                for cl in range(cin_g):
                    for kh in range(3):
                        for kw in range(3):
                            st = (dh + kh) * 4 + (dw + kw)
                            c = st * cin_g + cl
                            idx[r, c] = o * cin_g * 9 + cl * 9 + kh * 3 + kw
                            valid[r, c] = True
    return idx, valid


def _make_body(out_g, cin_g, H2, W2, S, inv_count, eps):
    def _body(xq_ref, weff_ref, gamma_ref, beta_ref, o_ref, p_ref):
        lane = jax.lax.broadcasted_iota(jnp.int32, (cin_g, S), 1)
        w2 = jax.lax.rem(lane, W2)
        h2 = jax.lax.rem(lane // W2, H2)
        for s in range(4):
            bs, a = _SPLIT[s]
            for t in range(4):
                bt, c = _SPLIT[t]
                st = s * 4 + t
                b = bs * 2 + bt
                src = xq_ref[0, b * cin_g:(b + 1) * cin_g, :]
                r = a * W2 + c
                if r != 0:
                    src = pltpu.roll(src, (-r) % S, axis=1)
                ok = None
                if a == -1:
                    ok = h2 >= 1
                elif a == 1:
                    ok = h2 <= H2 - 2
                if c == -1:
                    okw = w2 >= 1
                    ok = okw if ok is None else (ok & okw)
                elif c == 1:
                    okw = w2 <= W2 - 2
                    ok = okw if ok is None else (ok & okw)
                if ok is not None:
                    src = jnp.where(ok, src, 0.0)
                p_ref[st * cin_g:(st + 1) * cin_g, :] = src

        wf = weff_ref[0]
        acc = jnp.dot(wf, p_ref[...],
                      preferred_element_type=jnp.float32)

        rs = jnp.sum(acc, axis=1, keepdims=True)
        sch = (rs[0:out_g] + rs[out_g:2 * out_g]
               + rs[2 * out_g:3 * out_g] + rs[3 * out_g:4 * out_g])
        mean = sch * inv_count
        mean4 = jnp.concatenate([mean, mean, mean, mean], axis=0)
        d = acc - mean4
        rq = jnp.sum(d * d, axis=1, keepdims=True)
        var = (rq[0:out_g] + rq[out_g:2 * out_g]
               + rq[2 * out_g:3 * out_g] + rq[3 * out_g:4 * out_g]) * inv_count
        scale = gamma_ref[0] * jax.lax.rsqrt(var + eps)
        shift = beta_ref[0] - mean * scale
        scale4 = jnp.concatenate([scale, scale, scale, scale], axis=0)
        shift4 = jnp.concatenate([shift, shift, shift, shift], axis=0)

        y = jnp.maximum(acc * scale4 + shift4, 0.0)
        o_ref[0] = jnp.maximum(
            jnp.maximum(y[0:out_g], y[out_g:2 * out_g]),
            jnp.maximum(y[2 * out_g:3 * out_g], y[3 * out_g:4 * out_g]))

    return _body


def _forward(x, w, gamma, beta, groups, eps=1e-5):
    N, Cin, H, W = x.shape
    Cout = w.shape[0]
    H2, W2 = H // 2, W // 2
    G, cin_g, out_g = groups, Cin // groups, Cout // groups
    S = N * H2 * W2
    count = N * H * W

    xq = x.astype(jnp.float32).reshape(N, G, cin_g, H2, 2, W2, 2)
    xq = jnp.transpose(xq, (1, 4, 6, 2, 0, 3, 5))
    xq = xq.reshape(G, 4 * cin_g, S)

    idx, valid = _weff_index(out_g, cin_g)
    w_grp = w.astype(jnp.float32).reshape(G, out_g * cin_g * 9)
    w_eff = jnp.where(jnp.asarray(valid), w_grp[:, jnp.asarray(idx)], 0.0)

    gamma_c = gamma.astype(jnp.float32).reshape(G, out_g, 1)
    beta_c = beta.astype(jnp.float32).reshape(G, out_g, 1)

    body = _make_body(out_g, cin_g, H2, W2, S, 1.0 / float(count), eps)

    flops = 2 * S * 4 * Cout * cin_g * 16 + 8 * count * Cout
    bytes_accessed = xq.size * 4 + Cout * S * 4
    cost = pl.CostEstimate(flops=flops, transcendentals=Cout,
                           bytes_accessed=int(bytes_accessed))

    out_gm = pl.pallas_call(
        body,
        grid=(G,),
        in_specs=[
            pl.BlockSpec((1, 4 * cin_g, S), lambda g: (g, 0, 0)),
            pl.BlockSpec((1, 4 * out_g, cin_g * 16), lambda g: (g, 0, 0)),
            pl.BlockSpec((1, out_g, 1), lambda g: (g, 0, 0)),
            pl.BlockSpec((1, out_g, 1), lambda g: (g, 0, 0)),
        ],
        out_specs=pl.BlockSpec((1, out_g, S), lambda g: (g, 0, 0)),
        out_shape=jax.ShapeDtypeStruct((G, out_g, S), jnp.float32),
        scratch_shapes=[pltpu.VMEM((16 * cin_g, S), jnp.float32)],
        compiler_params=pltpu.CompilerParams(
            dimension_semantics=("parallel",),
            vmem_limit_bytes=60 * 1024 * 1024),
        cost_estimate=cost,
    )(xq, w_eff, gamma_c, beta_c)

    out = out_gm.reshape(Cout, N, H2, W2)
    return jnp.transpose(out, (1, 0, 2, 3))


def kernel(x, w, gamma, beta):
    return _forward(x, w, gamma, beta, groups=8)
